# baseline (speedup 1.0000x reference)
import functools

import jax
import jax.numpy as jnp
from jax import lax
from jax.experimental import pallas as pl
from jax.experimental.pallas import tpu as pltpu
from jax.experimental.pallas import tpu_sc as plsc

VOCAB = 9048
VEC = 100
KPAD = 128
D_MODEL = 1024
B, L = 64, 512
N = B * L

NC, NS = 2, 16
NW = NC * NS
S = 2
NSTRIPE = N // S
ROWS_PER_W = NSTRIPE // NW
CHUNK = 128
NCHUNK = ROWS_PER_W // CHUNK
NBUF = 4


def _sc_gather(table_pad, idx3):
    mesh = plsc.VectorSubcoreMesh(core_axis_name="c", subcore_axis_name="s")

    @functools.partial(
        pl.kernel,
        mesh=mesh,
        out_type=jax.ShapeDtypeStruct((NSTRIPE, KPAD), jnp.float32),
        scratch_types=[
            pltpu.VMEM((NCHUNK, CHUNK), jnp.int32),
        ]
        + [pltpu.VMEM((CHUNK, KPAD), jnp.float32) for _ in range(NBUF)]
        + [pltpu.SemaphoreType.DMA for _ in range(2 * NBUF)],
    )
    def k(table_hbm, idx_hbm, out_hbm, idx_v, *scratch):
        bufs = scratch[:NBUF]
        gsems = scratch[NBUF : 2 * NBUF]
        wsems = scratch[2 * NBUF :]
        wid = lax.axis_index("s") * NC + lax.axis_index("c")
        base = wid * ROWS_PER_W
        pltpu.sync_copy(idx_hbm.at[wid], idx_v)

        gcopies = [None] * NBUF
        wcopies = [None] * NBUF

        def fire_gather(c):
            s = c % NBUF
            gcopies[s] = pltpu.async_copy(table_hbm.at[idx_v.at[c]], bufs[s], gsems[s])

        for c in range(min(2, NCHUNK)):
            fire_gather(c)
        for c in range(NCHUNK):
            s = c % NBUF
            nxt = c + 2
            if nxt < NCHUNK:
                if nxt >= NBUF:
                    wcopies[nxt % NBUF].wait()
                fire_gather(nxt)
            gcopies[s].wait()
            wcopies[s] = pltpu.async_copy(
                bufs[s], out_hbm.at[pl.ds(base + c * CHUNK, CHUNK)], wsems[s]
            )
        for c in range(max(0, NCHUNK - NBUF), NCHUNK):
            wcopies[c % NBUF].wait()

    return k(table_pad, idx3)


BM = 4096
BLOCKS_PER_STRIPE = NSTRIPE // BM


def _tc_project_stripe(stripe, x, w_pad, bias2d, prev_out):
    def body(x_ref, w_ref, b_ref, *rest):
        o_ref = rest[-1]
        o_ref[...] = (
            jnp.dot(x_ref[...], w_ref[...], preferred_element_type=jnp.float32)
            + b_ref[...]
        )

    in_specs = [
        pl.BlockSpec((BM, KPAD), lambda i: (i, 0)),
        pl.BlockSpec((KPAD, D_MODEL), lambda i: (0, 0)),
        pl.BlockSpec((1, D_MODEL), lambda i: (0, 0)),
    ]
    args = [x, w_pad, bias2d]
    aliases = {}
    if prev_out is not None:
        in_specs.append(pl.BlockSpec(memory_space=pl.ANY))
        args.append(prev_out)
        aliases = {3: 0}

    return pl.pallas_call(
        body,
        grid=(BLOCKS_PER_STRIPE,),
        in_specs=in_specs,
        out_specs=pl.BlockSpec(
            (BM, D_MODEL), lambda i, _s=stripe: (_s * BLOCKS_PER_STRIPE + i, 0)
        ),
        out_shape=jax.ShapeDtypeStruct((N, D_MODEL), jnp.float32),
        input_output_aliases=aliases,
    )(*args)


def kernel(protX, table, W, b):
    idx4 = protX.reshape(-1).astype(jnp.int32).reshape(S, NW, NCHUNK, CHUNK)
    table_pad = jnp.pad(table, ((0, 0), (0, KPAD - VEC)))
    w_pad = jnp.pad(W, ((0, KPAD - VEC), (0, 0)))
    bias2d = b.reshape(1, D_MODEL)
    gathered = [_sc_gather(table_pad, idx4[s]) for s in range(S)]
    out = None
    for s in range(S):
        out = _tc_project_stripe(s, gathered[s], w_pad, bias2d, out)
    return out.reshape(B, L, D_MODEL)


# speedup vs baseline: 1.0341x; 1.0341x over previous
import functools

import jax
import jax.numpy as jnp
from jax import lax
from jax.experimental import pallas as pl
from jax.experimental.pallas import tpu as pltpu
from jax.experimental.pallas import tpu_sc as plsc

VOCAB = 9048
VEC = 100
KPAD = 128
D_MODEL = 1024
B, L = 64, 512
N = B * L

NC, NS = 2, 16
NW = NC * NS
S = 1
NSTRIPE = N // S
ROWS_PER_W = NSTRIPE // NW
CHUNK = 128
NCHUNK = ROWS_PER_W // CHUNK
NBUF = 6


def _sc_gather(table_pad, idx3):
    mesh = plsc.VectorSubcoreMesh(core_axis_name="c", subcore_axis_name="s")

    @functools.partial(
        pl.kernel,
        mesh=mesh,
        out_type=jax.ShapeDtypeStruct((NSTRIPE, KPAD), jnp.float32),
        scratch_types=[
            pltpu.VMEM((NCHUNK, CHUNK), jnp.int32),
        ]
        + [pltpu.VMEM((CHUNK, KPAD), jnp.float32) for _ in range(NBUF)]
        + [pltpu.SemaphoreType.DMA for _ in range(2 * NBUF)],
    )
    def k(table_hbm, idx_hbm, out_hbm, idx_v, *scratch):
        bufs = scratch[:NBUF]
        gsems = scratch[NBUF : 2 * NBUF]
        wsems = scratch[2 * NBUF :]
        wid = lax.axis_index("s") * NC + lax.axis_index("c")
        base = wid * ROWS_PER_W
        pltpu.sync_copy(idx_hbm.at[wid], idx_v)

        gcopies = [None] * NBUF
        wcopies = [None] * NBUF

        def fire_gather(c):
            s = c % NBUF
            gcopies[s] = pltpu.async_copy(table_hbm.at[idx_v.at[c]], bufs[s], gsems[s])

        for c in range(min(3, NCHUNK)):
            fire_gather(c)
        for c in range(NCHUNK):
            s = c % NBUF
            nxt = c + 3
            if nxt < NCHUNK:
                if nxt >= NBUF:
                    wcopies[nxt % NBUF].wait()
                fire_gather(nxt)
            gcopies[s].wait()
            wcopies[s] = pltpu.async_copy(
                bufs[s], out_hbm.at[pl.ds(base + c * CHUNK, CHUNK)], wsems[s]
            )
        for c in range(max(0, NCHUNK - NBUF), NCHUNK):
            wcopies[c % NBUF].wait()

    return k(table_pad, idx3)


BM = 4096
BLOCKS_PER_STRIPE = NSTRIPE // BM


def _tc_project_stripe(stripe, x, w_pad, bias2d, prev_out):
    def body(x_ref, w_ref, b_ref, *rest):
        o_ref = rest[-1]
        o_ref[...] = (
            jnp.dot(x_ref[...], w_ref[...], preferred_element_type=jnp.float32)
            + b_ref[...]
        )

    in_specs = [
        pl.BlockSpec((BM, KPAD), lambda i: (i, 0)),
        pl.BlockSpec((KPAD, D_MODEL), lambda i: (0, 0)),
        pl.BlockSpec((1, D_MODEL), lambda i: (0, 0)),
    ]
    args = [x, w_pad, bias2d]
    aliases = {}
    if prev_out is not None:
        in_specs.append(pl.BlockSpec(memory_space=pl.ANY))
        args.append(prev_out)
        aliases = {3: 0}

    return pl.pallas_call(
        body,
        grid=(BLOCKS_PER_STRIPE,),
        in_specs=in_specs,
        out_specs=pl.BlockSpec(
            (BM, D_MODEL), lambda i, _s=stripe: (_s * BLOCKS_PER_STRIPE + i, 0)
        ),
        out_shape=jax.ShapeDtypeStruct((N, D_MODEL), jnp.float32),
        input_output_aliases=aliases,
    )(*args)


def kernel(protX, table, W, b):
    idx4 = protX.reshape(-1).astype(jnp.int32).reshape(S, NW, NCHUNK, CHUNK)
    table_pad = jnp.pad(table, ((0, 0), (0, KPAD - VEC)))
    w_pad = jnp.pad(W, ((0, KPAD - VEC), (0, 0)))
    bias2d = b.reshape(1, D_MODEL)
    gathered = [_sc_gather(table_pad, idx4[s]) for s in range(S)]
    out = None
    for s in range(S):
        out = _tc_project_stripe(s, gathered[s], w_pad, bias2d, out)
    return out.reshape(B, L, D_MODEL)
